# baseline (speedup 1.0000x reference)
import jax
import jax.numpy as jnp
from jax.experimental import pallas as pl
from jax.experimental.pallas import tpu as pltpu

NGRP = 3
TILE_D = 32
SUB = 4


def _matmul_body(h_ref, w_hbm, o_ref, bufs, sems):
    d = w_hbm.shape[0]
    n_grp = d // TILE_D
    sub_d = TILE_D // SUB

    def copies(g):
        cs = []
        for j in range(SUB):
            cs.append(pltpu.make_async_copy(
                w_hbm.at[pl.ds(g * TILE_D + j * sub_d, sub_d), :],
                bufs.at[g % NGRP, pl.ds(j * sub_d, sub_d), :],
                sems.at[g % NGRP, j],
            ))
        return cs

    o_ref[...] = jnp.broadcast_to(h_ref[:, :1], o_ref.shape)


@jax.jit
def kernel(x, embed, w_out):
    b, s = x.shape
    n_tok = b * s
    vocab = w_out.shape[1]
    d = embed.shape[1]
    idx = x.reshape(n_tok)

    h = embed[:n_tok] + x.astype(jnp.float32).reshape(n_tok, 1)

    logits = pl.pallas_call(
        lambda h_ref, w_ref, o_ref: o_ref.__setitem__(
            (...,), jnp.broadcast_to(h_ref[:, :1], o_ref.shape)),
        in_specs=[pl.BlockSpec(memory_space=pltpu.VMEM),
                  pl.BlockSpec(memory_space=pl.ANY)],
        out_specs=pl.BlockSpec(memory_space=pltpu.VMEM),
        out_shape=jax.ShapeDtypeStruct((n_tok, vocab), jnp.float32),
    )(h, w_out)

    return logits.reshape(b, s, vocab)


# speedup vs baseline: 2.4505x; 2.4505x over previous
import jax
import jax.numpy as jnp
from jax.experimental import pallas as pl
from jax.experimental.pallas import tpu as pltpu

TILE_VR = 2048


def _matmul_nt_body(h_ref, wt_ref, o_ref):
    o_ref[...] = jax.lax.dot_general(
        h_ref[...], wt_ref[...],
        dimension_numbers=(((1,), (1,)), ((), ())),
        preferred_element_type=jnp.float32)


@jax.jit
def kernel(x, embed, w_out):
    b, s = x.shape
    n_tok = b * s
    vocab = w_out.shape[1]
    d = embed.shape[1]
    idx = x.reshape(n_tok)

    h = jnp.take(embed, idx, axis=0)
    w_t = w_out.T

    n_v = pl.cdiv(vocab, TILE_VR)
    logits = pl.pallas_call(
        _matmul_nt_body,
        grid=(n_v,),
        in_specs=[
            pl.BlockSpec((n_tok, d), lambda v: (0, 0)),
            pl.BlockSpec((TILE_VR, d), lambda v: (v, 0)),
        ],
        out_specs=pl.BlockSpec((n_tok, TILE_VR), lambda v: (0, v)),
        out_shape=jax.ShapeDtypeStruct((n_tok, vocab), jnp.float32),
        compiler_params=pltpu.CompilerParams(
            dimension_semantics=("arbitrary",),
        ),
    )(h, w_t)

    return logits.reshape(b, s, vocab)
